# baseline (speedup 1.0000x reference)
import functools

import jax
import jax.numpy as jnp
from jax import lax
from jax.experimental import pallas as pl
from jax.experimental.pallas import tpu as pltpu
from jax.experimental.pallas import tpu_sc as plsc

N = 10000
E = 320000
D = 128

NPAD = 10240
SLICE = NPAD // 16
CHUNK = 128
NW = 32
CPW = 80
EPW = CPW * CHUNK
EPAD = NW * EPW

_mesh = plsc.VectorSubcoreMesh(core_axis_name="c", subcore_axis_name="s")


@functools.partial(
    pl.kernel,
    mesh=_mesh,
    out_type=jax.ShapeDtypeStruct((2, NPAD), jnp.float32),
    scratch_types=[
        pltpu.VMEM((EPW,), jnp.int32),
        pltpu.VMEM((CHUNK,), jnp.float32),
        pltpu.VMEM_SHARED((NPAD,), jnp.float32),
        pltpu.SemaphoreType.DMA,
    ],
)
def _sc_degree(rows_hbm, zeros1_hbm, deg_out, rid_v, ones_v, deg_sh, sem):
    c = lax.axis_index("c")
    s = lax.axis_index("s")
    wid = s * 2 + c
    pltpu.sync_copy(rows_hbm.at[pl.ds(wid * EPW, EPW)], rid_v)
    for i in range(CHUNK // 16):
        ones_v[pl.ds(i * 16, 16)] = jnp.ones((16,), jnp.float32)
    pltpu.sync_copy(zeros1_hbm.at[pl.ds(s * SLICE, SLICE)],
                    deg_sh.at[pl.ds(s * SLICE, SLICE)])
    plsc.subcore_barrier()

    def body(j, carry):
        idx = rid_v.at[pl.ds(j * CHUNK, CHUNK)]
        pltpu.async_copy(ones_v, deg_sh.at[idx], sem, add=True)
        return carry

    lax.fori_loop(0, CPW, body, 0)

    def drain(j, carry):
        idx = rid_v.at[pl.ds(j * CHUNK, CHUNK)]
        pltpu.make_async_copy(ones_v, deg_sh.at[idx], sem).wait()
        return carry

    lax.fori_loop(0, CPW, drain, 0)
    plsc.subcore_barrier()
    pltpu.sync_copy(deg_sh.at[pl.ds(s * SLICE, SLICE)],
                    deg_out.at[c, pl.ds(s * SLICE, SLICE)])


@functools.partial(
    pl.kernel,
    mesh=_mesh,
    out_type=jax.ShapeDtypeStruct((2, NPAD, D), jnp.float32),
    scratch_types=[
        pltpu.VMEM((EPW,), jnp.int32),
        pltpu.VMEM((EPW,), jnp.int32),
        pltpu.VMEM((CHUNK, D), jnp.float32),
        pltpu.VMEM_SHARED((NPAD, D), jnp.float32),
        pltpu.SemaphoreType.DMA,
    ],
)
def _sc_aggregate(u_hbm, cols_hbm, rows_hbm, zeros2_hbm, s_out,
                  cid_v, rid_v, buf, s_sh, gsem):
    c = lax.axis_index("c")
    s = lax.axis_index("s")
    wid = s * 2 + c
    pltpu.sync_copy(cols_hbm.at[pl.ds(wid * EPW, EPW)], cid_v)
    pltpu.sync_copy(rows_hbm.at[pl.ds(wid * EPW, EPW)], rid_v)
    pltpu.sync_copy(zeros2_hbm.at[pl.ds(s * SLICE, SLICE)],
                    s_sh.at[pl.ds(s * SLICE, SLICE)])
    plsc.subcore_barrier()

    def body(j, carry):
        cid = cid_v.at[pl.ds(j * CHUNK, CHUNK)]
        rid = rid_v.at[pl.ds(j * CHUNK, CHUNK)]
        pltpu.async_copy(u_hbm.at[cid], buf, gsem).wait()
        pltpu.sync_copy(buf, s_sh.at[rid], add=True)
        return carry

    lax.fori_loop(0, CPW, body, 0)
    plsc.subcore_barrier()
    pltpu.sync_copy(s_sh.at[pl.ds(s * SLICE, SLICE)],
                    s_out.at[c, pl.ds(s * SLICE, SLICE)])


def _tc_scale_body(deg_ref, x_ref, u_ref, dinv_ref):
    deg = deg_ref[0] + deg_ref[1]
    dinv = lax.rsqrt(deg)
    dinv_ref[...] = dinv
    u_ref[...] = dinv * x_ref[...]


def _tc_final_body(s_ref, u_ref, dinv_ref, w_ref, b_ref, out_ref):
    agg = s_ref[0] + s_ref[1] + u_ref[...]
    h = dinv_ref[...] * agg
    hw = lax.dot_general(h, w_ref[...], (((1,), (1,)), ((), ())),
                         preferred_element_type=jnp.float32)
    out_ref[...] = jnp.maximum(hw + b_ref[...], 0.0)


BLK = 2000
GRID = N // BLK


def kernel(x, edge_index, W, b):
    pad = EPAD - E
    pad_rows = N + jnp.arange(pad, dtype=jnp.int32) % (NPAD - N)
    rows = jnp.concatenate([edge_index[0], pad_rows])
    cols = jnp.concatenate([edge_index[1], jnp.zeros((pad,), jnp.int32)])
    zeros1 = jnp.zeros((NPAD,), jnp.float32)
    zeros2 = jnp.zeros((NPAD, D), jnp.float32)

    deg_parts = _sc_degree(rows, zeros1).reshape(2, NPAD, 1)

    u, dinv = pl.pallas_call(
        _tc_scale_body,
        grid=(GRID,),
        in_specs=[
            pl.BlockSpec((2, BLK, 1), lambda i: (0, i, 0)),
            pl.BlockSpec((BLK, D), lambda i: (i, 0)),
        ],
        out_specs=[
            pl.BlockSpec((BLK, D), lambda i: (i, 0)),
            pl.BlockSpec((BLK, 1), lambda i: (i, 0)),
        ],
        out_shape=[
            jax.ShapeDtypeStruct((N, D), jnp.float32),
            jax.ShapeDtypeStruct((N, 1), jnp.float32),
        ],
    )(deg_parts, x)

    s_parts = _sc_aggregate(u, cols, rows, zeros2)

    out = pl.pallas_call(
        _tc_final_body,
        grid=(GRID,),
        in_specs=[
            pl.BlockSpec((2, BLK, D), lambda i: (0, i, 0)),
            pl.BlockSpec((BLK, D), lambda i: (i, 0)),
            pl.BlockSpec((BLK, 1), lambda i: (i, 0)),
            pl.BlockSpec((D, D), lambda i: (0, 0)),
            pl.BlockSpec((1, D), lambda i: (0, 0)),
        ],
        out_specs=pl.BlockSpec((BLK, D), lambda i: (i, 0)),
        out_shape=jax.ShapeDtypeStruct((N, D), jnp.float32),
    )(s_parts, u, dinv, W, b.reshape(1, D))

    return out


# speedup vs baseline: 1.0710x; 1.0710x over previous
import functools

import jax
import jax.numpy as jnp
from jax import lax
from jax.experimental import pallas as pl
from jax.experimental.pallas import tpu as pltpu
from jax.experimental.pallas import tpu_sc as plsc

N = 10000
E = 320000
D = 128

NPAD = 10240
SLICE = NPAD // 16
CHUNK = 128
NW = 32
CPW = 80
EPW = CPW * CHUNK
EPAD = NW * EPW

_mesh = plsc.VectorSubcoreMesh(core_axis_name="c", subcore_axis_name="s")


@functools.partial(
    pl.kernel,
    mesh=_mesh,
    out_type=jax.ShapeDtypeStruct((2, NPAD), jnp.float32),
    scratch_types=[
        pltpu.VMEM((EPW,), jnp.int32),
        pltpu.VMEM((CHUNK,), jnp.float32),
        pltpu.VMEM_SHARED((NPAD,), jnp.float32),
        pltpu.SemaphoreType.DMA,
    ],
)
def _sc_degree(rows_hbm, zeros1_hbm, deg_out, rid_v, ones_v, deg_sh, sem):
    c = lax.axis_index("c")
    s = lax.axis_index("s")
    wid = s * 2 + c
    pltpu.sync_copy(rows_hbm.at[pl.ds(wid * EPW, EPW)], rid_v)
    for i in range(CHUNK // 16):
        ones_v[pl.ds(i * 16, 16)] = jnp.ones((16,), jnp.float32)
    pltpu.sync_copy(zeros1_hbm.at[pl.ds(s * SLICE, SLICE)],
                    deg_sh.at[pl.ds(s * SLICE, SLICE)])
    plsc.subcore_barrier()

    def body(j, carry):
        idx = rid_v.at[pl.ds(j * CHUNK, CHUNK)]
        pltpu.async_copy(ones_v, deg_sh.at[idx], sem, add=True)
        return carry

    lax.fori_loop(0, CPW, body, 0)

    def drain(j, carry):
        idx = rid_v.at[pl.ds(j * CHUNK, CHUNK)]
        pltpu.make_async_copy(ones_v, deg_sh.at[idx], sem).wait()
        return carry

    lax.fori_loop(0, CPW, drain, 0)
    plsc.subcore_barrier()
    pltpu.sync_copy(deg_sh.at[pl.ds(s * SLICE, SLICE)],
                    deg_out.at[c, pl.ds(s * SLICE, SLICE)])


@functools.partial(
    pl.kernel,
    mesh=_mesh,
    out_type=jax.ShapeDtypeStruct((2, NPAD, D), jnp.float32),
    scratch_types=[
        pltpu.VMEM((CHUNK,), jnp.int32),
        pltpu.VMEM((CHUNK,), jnp.int32),
        pltpu.VMEM((CHUNK, D), jnp.float32),
        pltpu.VMEM_SHARED((NPAD, D), jnp.float32),
        pltpu.SemaphoreType.DMA,
    ],
)
def _sc_aggregate(u_hbm, cols_hbm, rows_hbm, zeros2_hbm, s_out,
                  cid_v, rid_v, buf, s_sh, gsem):
    c = lax.axis_index("c")
    s = lax.axis_index("s")
    wid = s * 2 + c
    pltpu.sync_copy(zeros2_hbm.at[pl.ds(s * SLICE, SLICE)],
                    s_sh.at[pl.ds(s * SLICE, SLICE)])
    plsc.subcore_barrier()

    def body(j, carry):
        chunk = wid + NW * j
        pltpu.sync_copy(cols_hbm.at[pl.ds(chunk * CHUNK, CHUNK)], cid_v)
        pltpu.sync_copy(rows_hbm.at[pl.ds(chunk * CHUNK, CHUNK)], rid_v)
        pltpu.async_copy(u_hbm.at[cid_v], buf, gsem).wait()
        pltpu.sync_copy(buf, s_sh.at[rid_v], add=True)
        return carry

    lax.fori_loop(0, CPW, body, 0)
    plsc.subcore_barrier()
    pltpu.sync_copy(s_sh.at[pl.ds(s * SLICE, SLICE)],
                    s_out.at[c, pl.ds(s * SLICE, SLICE)])


def _tc_scale_body(deg_ref, x_ref, u_ref, dinv_ref):
    deg = deg_ref[0] + deg_ref[1]
    dinv = lax.rsqrt(deg)
    dinv_ref[...] = dinv
    u_ref[...] = dinv * x_ref[...]


def _tc_final_body(s_ref, u_ref, dinv_ref, w_ref, b_ref, out_ref):
    agg = s_ref[0] + s_ref[1] + u_ref[...]
    h = dinv_ref[...] * agg
    hw = lax.dot_general(h, w_ref[...], (((1,), (1,)), ((), ())),
                         preferred_element_type=jnp.float32)
    out_ref[...] = jnp.maximum(hw + b_ref[...], 0.0)


BLK = 2000
GRID = N // BLK


def kernel(x, edge_index, W, b):
    pad = EPAD - E
    pad_rows = N + jnp.arange(pad, dtype=jnp.int32) % (NPAD - N)
    rows = jnp.concatenate([edge_index[0], pad_rows])
    cols = jnp.concatenate([edge_index[1], jnp.zeros((pad,), jnp.int32)])
    zeros1 = jnp.zeros((NPAD,), jnp.float32)
    zeros2 = jnp.zeros((NPAD, D), jnp.float32)

    deg_parts = _sc_degree(rows, zeros1).reshape(2, NPAD, 1)

    u, dinv = pl.pallas_call(
        _tc_scale_body,
        grid=(GRID,),
        in_specs=[
            pl.BlockSpec((2, BLK, 1), lambda i: (0, i, 0)),
            pl.BlockSpec((BLK, D), lambda i: (i, 0)),
        ],
        out_specs=[
            pl.BlockSpec((BLK, D), lambda i: (i, 0)),
            pl.BlockSpec((BLK, 1), lambda i: (i, 0)),
        ],
        out_shape=[
            jax.ShapeDtypeStruct((N, D), jnp.float32),
            jax.ShapeDtypeStruct((N, 1), jnp.float32),
        ],
    )(deg_parts, x)

    s_parts = _sc_aggregate(u, cols, rows, zeros2)

    out = pl.pallas_call(
        _tc_final_body,
        grid=(GRID,),
        in_specs=[
            pl.BlockSpec((2, BLK, D), lambda i: (0, i, 0)),
            pl.BlockSpec((BLK, D), lambda i: (i, 0)),
            pl.BlockSpec((BLK, 1), lambda i: (i, 0)),
            pl.BlockSpec((D, D), lambda i: (0, 0)),
            pl.BlockSpec((1, D), lambda i: (0, 0)),
        ],
        out_specs=pl.BlockSpec((BLK, D), lambda i: (i, 0)),
        out_shape=jax.ShapeDtypeStruct((N, D), jnp.float32),
    )(s_parts, u, dinv, W, b.reshape(1, D))

    return out


# speedup vs baseline: 1.9725x; 1.8417x over previous
import functools

import jax
import jax.numpy as jnp
from jax import lax
from jax.experimental import pallas as pl
from jax.experimental.pallas import tpu as pltpu
from jax.experimental.pallas import tpu_sc as plsc

N = 10000
E = 320000
D = 128

NPAD = 10240
SLICE = NPAD // 16
CHUNK = 128
NW = 32
CPW = 80
EPW = CPW * CHUNK
EPAD = NW * EPW

_mesh = plsc.VectorSubcoreMesh(core_axis_name="c", subcore_axis_name="s")


@functools.partial(
    pl.kernel,
    mesh=_mesh,
    out_type=jax.ShapeDtypeStruct((2, NPAD), jnp.float32),
    scratch_types=[
        pltpu.VMEM((EPW,), jnp.int32),
        pltpu.VMEM((CHUNK,), jnp.float32),
        pltpu.VMEM_SHARED((NPAD,), jnp.float32),
        pltpu.SemaphoreType.DMA,
    ],
)
def _sc_degree(rows_hbm, zeros1_hbm, deg_out, rid_v, ones_v, deg_sh, sem):
    c = lax.axis_index("c")
    s = lax.axis_index("s")
    wid = s * 2 + c
    pltpu.sync_copy(rows_hbm.at[pl.ds(wid * EPW, EPW)], rid_v)
    for i in range(CHUNK // 16):
        ones_v[pl.ds(i * 16, 16)] = jnp.ones((16,), jnp.float32)
    pltpu.sync_copy(zeros1_hbm.at[pl.ds(s * SLICE, SLICE)],
                    deg_sh.at[pl.ds(s * SLICE, SLICE)])
    plsc.subcore_barrier()

    def body(j, carry):
        idx = rid_v.at[pl.ds(j * CHUNK, CHUNK)]
        pltpu.async_copy(ones_v, deg_sh.at[idx], sem, add=True)
        return carry

    lax.fori_loop(0, CPW, body, 0)

    def drain(j, carry):
        idx = rid_v.at[pl.ds(j * CHUNK, CHUNK)]
        pltpu.make_async_copy(ones_v, deg_sh.at[idx], sem).wait()
        return carry

    lax.fori_loop(0, CPW, drain, 0)
    plsc.subcore_barrier()
    pltpu.sync_copy(deg_sh.at[pl.ds(s * SLICE, SLICE)],
                    deg_out.at[c, pl.ds(s * SLICE, SLICE)])


@functools.partial(
    pl.kernel,
    mesh=_mesh,
    out_type=jax.ShapeDtypeStruct((2, NPAD, D), jnp.float32),
    scratch_types=[
        pltpu.VMEM((CHUNK,), jnp.int32),
        pltpu.VMEM((CHUNK,), jnp.int32),
        pltpu.VMEM((CHUNK, D), jnp.float32),
        pltpu.VMEM_SHARED((NPAD, D), jnp.float32),
        pltpu.SemaphoreType.DMA,
    ],
)
def _sc_aggregate(u_hbm, cols_hbm, rows_hbm, zeros2_hbm, s_out,
                  cid_v, rid_v, buf, s_sh, gsem):
    c = lax.axis_index("c")
    s = lax.axis_index("s")
    wid = s * 2 + c
    pltpu.sync_copy(zeros2_hbm.at[pl.ds(s * SLICE, SLICE)],
                    s_sh.at[pl.ds(s * SLICE, SLICE)])
    plsc.subcore_barrier()

    def body(j, carry):
        chunk = wid + NW * j
        pltpu.sync_copy(cols_hbm.at[pl.ds(chunk * CHUNK, CHUNK)], cid_v)
        pltpu.sync_copy(rows_hbm.at[pl.ds(chunk * CHUNK, CHUNK)], rid_v)
        pltpu.async_copy(u_hbm.at[cid_v], buf, gsem).wait()
        pltpu.sync_copy(buf, s_sh.at[rid_v], add=True)
        return carry

    lax.fori_loop(0, CPW, body, 0)
    plsc.subcore_barrier()
    pltpu.sync_copy(s_sh.at[pl.ds(s * SLICE, SLICE)],
                    s_out.at[c, pl.ds(s * SLICE, SLICE)])


def _tc_scale_body(deg_ref, x_ref, u_ref, dinv_ref):
    deg = deg_ref[0] + deg_ref[1]
    dinv = lax.rsqrt(deg)
    dinv_ref[...] = dinv
    u_ref[...] = dinv * x_ref[...]


def _tc_final_body(s_ref, u_ref, dinv_ref, w_ref, b_ref, out_ref):
    agg = s_ref[0] + s_ref[1] + u_ref[...]
    h = dinv_ref[...] * agg
    hw = lax.dot_general(h, w_ref[...], (((1,), (1,)), ((), ())),
                         preferred_element_type=jnp.float32)
    out_ref[...] = jnp.maximum(hw + b_ref[...], 0.0)


BLK = 2000
GRID = N // BLK


def kernel(x, edge_index, W, b):
    pad = EPAD - E
    ar = jnp.arange(pad, dtype=jnp.int32)
    rows_h = jnp.concatenate([edge_index[0], N + ar % (NPAD - N)])
    rows_a = jnp.concatenate([edge_index[0], ar % N])
    cols_a = jnp.concatenate([edge_index[1], N + ar % (NPAD - N)])
    zeros1 = jnp.zeros((NPAD,), jnp.float32)
    zeros2 = jnp.zeros((NPAD, D), jnp.float32)

    deg_parts = _sc_degree(rows_h, zeros1).reshape(2, NPAD, 1)

    u, dinv = pl.pallas_call(
        _tc_scale_body,
        grid=(GRID,),
        in_specs=[
            pl.BlockSpec((2, BLK, 1), lambda i: (0, i, 0)),
            pl.BlockSpec((BLK, D), lambda i: (i, 0)),
        ],
        out_specs=[
            pl.BlockSpec((BLK, D), lambda i: (i, 0)),
            pl.BlockSpec((BLK, 1), lambda i: (i, 0)),
        ],
        out_shape=[
            jax.ShapeDtypeStruct((N, D), jnp.float32),
            jax.ShapeDtypeStruct((N, 1), jnp.float32),
        ],
    )(deg_parts, x)

    u_pad = jnp.concatenate([u, jnp.zeros((NPAD - N, D), jnp.float32)])
    s_parts = _sc_aggregate(u_pad, cols_a, rows_a, zeros2)

    out = pl.pallas_call(
        _tc_final_body,
        grid=(GRID,),
        in_specs=[
            pl.BlockSpec((2, BLK, D), lambda i: (0, i, 0)),
            pl.BlockSpec((BLK, D), lambda i: (i, 0)),
            pl.BlockSpec((BLK, 1), lambda i: (i, 0)),
            pl.BlockSpec((D, D), lambda i: (0, 0)),
            pl.BlockSpec((1, D), lambda i: (0, 0)),
        ],
        out_specs=pl.BlockSpec((BLK, D), lambda i: (i, 0)),
        out_shape=jax.ShapeDtypeStruct((N, D), jnp.float32),
    )(s_parts, u, dinv, W, b.reshape(1, D))

    return out


# speedup vs baseline: 3.1060x; 1.5747x over previous
import functools

import jax
import jax.numpy as jnp
from jax import lax
from jax.experimental import pallas as pl
from jax.experimental.pallas import tpu as pltpu
from jax.experimental.pallas import tpu_sc as plsc

N = 10000
E = 320000
D = 128

NPAD = 10240
SLICE = NPAD // 16
CHUNK = 128
NW = 32
CPW = 80
EPW = CPW * CHUNK
EPAD = NW * EPW
IB = 8
NB = CPW // IB

_mesh = plsc.VectorSubcoreMesh(core_axis_name="c", subcore_axis_name="s")


@functools.partial(
    pl.kernel,
    mesh=_mesh,
    out_type=jax.ShapeDtypeStruct((2, NPAD), jnp.float32),
    scratch_types=[
        pltpu.VMEM((EPW,), jnp.int32),
        pltpu.VMEM((CHUNK,), jnp.float32),
        pltpu.VMEM_SHARED((NPAD,), jnp.float32),
        pltpu.SemaphoreType.DMA,
    ],
)
def _sc_degree(rows_hbm, zeros1_hbm, deg_out, rid_v, ones_v, deg_sh, sem):
    c = lax.axis_index("c")
    s = lax.axis_index("s")
    wid = s * 2 + c
    pltpu.sync_copy(rows_hbm.at[pl.ds(wid * EPW, EPW)], rid_v)
    for i in range(CHUNK // 16):
        ones_v[pl.ds(i * 16, 16)] = jnp.ones((16,), jnp.float32)
    pltpu.sync_copy(zeros1_hbm.at[pl.ds(s * SLICE, SLICE)],
                    deg_sh.at[pl.ds(s * SLICE, SLICE)])
    plsc.subcore_barrier()

    def body(j, carry):
        idx = rid_v.at[pl.ds(j * CHUNK, CHUNK)]
        pltpu.async_copy(ones_v, deg_sh.at[idx], sem, add=True)
        return carry

    lax.fori_loop(0, CPW, body, 0)

    def drain(j, carry):
        idx = rid_v.at[pl.ds(j * CHUNK, CHUNK)]
        pltpu.make_async_copy(ones_v, deg_sh.at[idx], sem).wait()
        return carry

    lax.fori_loop(0, CPW, drain, 0)
    plsc.subcore_barrier()
    pltpu.sync_copy(deg_sh.at[pl.ds(s * SLICE, SLICE)],
                    deg_out.at[c, pl.ds(s * SLICE, SLICE)])


@functools.partial(
    pl.kernel,
    mesh=_mesh,
    out_type=jax.ShapeDtypeStruct((2, NPAD, D), jnp.float32),
    scratch_types=[
        pltpu.VMEM((IB * CHUNK,), jnp.int32),
        pltpu.VMEM((IB * CHUNK,), jnp.int32),
        pltpu.VMEM((IB * CHUNK,), jnp.int32),
        pltpu.VMEM((IB * CHUNK,), jnp.int32),
        pltpu.VMEM((CHUNK, D), jnp.float32),
        pltpu.VMEM((CHUNK, D), jnp.float32),
        pltpu.VMEM_SHARED((NPAD, D), jnp.float32),
    ] + [pltpu.SemaphoreType.DMA] * 4,
)
def _sc_aggregate(u_hbm, cols_hbm, rows_hbm, zeros2_hbm, s_out,
                  cidb0, cidb1, ridb0, ridb1, bufa, bufb, s_sh,
                  isem0, isem1, gsema, gsemb):
    cidb = (cidb0, cidb1)
    ridb = (ridb0, ridb1)
    bufs = (bufa, bufb)
    gsem = (gsema, gsemb)
    isem = (isem0, isem1)
    c = lax.axis_index("c")
    s = lax.axis_index("s")
    wid = s * 2 + c
    start = wid * EPW
    IBE = IB * CHUNK

    pltpu.sync_copy(cols_hbm.at[pl.ds(start, IBE)], cidb0)
    pltpu.sync_copy(rows_hbm.at[pl.ds(start, IBE)], ridb0)
    pltpu.sync_copy(zeros2_hbm.at[pl.ds(s * SLICE, SLICE)],
                    s_sh.at[pl.ds(s * SLICE, SLICE)])
    plsc.subcore_barrier()
    pltpu.async_copy(u_hbm.at[cidb0.at[pl.ds(0, CHUNK)]], bufa, gsema)

    def pair_body(p, carry):
        for bb in range(2):
            b = 2 * p + bb
            nbb = 1 - bb
            nslab = pl.ds(start + (b + 1) * IBE, IBE)
            for t in range(IB):
                sl = t % 2
                slp = 1 - sl
                cid = cidb[bb].at[pl.ds(t * CHUNK, CHUNK)]
                rid = ridb[bb].at[pl.ds(t * CHUNK, CHUNK)]

                pltpu.make_async_copy(u_hbm.at[cid], bufs[sl],
                                      gsem[sl]).wait()

                if t == IB - 1:
                    @pl.when(b < NB - 1)
                    def _():
                        pltpu.make_async_copy(cols_hbm.at[nslab],
                                              cidb[nbb], isem[nbb]).wait()
                        pltpu.make_async_copy(rows_hbm.at[nslab],
                                              ridb[nbb], isem[nbb]).wait()
                        pltpu.async_copy(
                            u_hbm.at[cidb[nbb].at[pl.ds(0, CHUNK)]],
                            bufs[slp], gsem[slp])
                else:
                    pltpu.async_copy(
                        u_hbm.at[cidb[bb].at[pl.ds((t + 1) * CHUNK, CHUNK)]],
                        bufs[slp], gsem[slp])

                pltpu.sync_copy(bufs[sl], s_sh.at[rid], add=True)

                if t == 2:
                    @pl.when(b < NB - 1)
                    def _():
                        pltpu.async_copy(cols_hbm.at[nslab], cidb[nbb],
                                         isem[nbb])
                        pltpu.async_copy(rows_hbm.at[nslab], ridb[nbb],
                                         isem[nbb])
        return carry

    lax.fori_loop(0, NB // 2, pair_body, 0)
    plsc.subcore_barrier()
    pltpu.sync_copy(s_sh.at[pl.ds(s * SLICE, SLICE)],
                    s_out.at[c, pl.ds(s * SLICE, SLICE)])


def _tc_scale_body(deg_ref, x_ref, u_ref, dinv_ref):
    deg = deg_ref[0] + deg_ref[1]
    dinv = lax.rsqrt(deg)
    dinv_ref[...] = dinv
    u_ref[...] = dinv * x_ref[...]


def _tc_final_body(s_ref, u_ref, dinv_ref, w_ref, b_ref, out_ref):
    agg = s_ref[0] + s_ref[1] + u_ref[...]
    h = dinv_ref[...] * agg
    hw = lax.dot_general(h, w_ref[...], (((1,), (1,)), ((), ())),
                         preferred_element_type=jnp.float32)
    out_ref[...] = jnp.maximum(hw + b_ref[...], 0.0)


BLK = 2000
GRID = N // BLK


def kernel(x, edge_index, W, b):
    pad = EPAD - E
    ar = jnp.arange(pad, dtype=jnp.int32)
    rows_h = jnp.concatenate([edge_index[0], N + ar % (NPAD - N)])
    rows_a = jnp.concatenate([edge_index[0], ar % N])
    cols_a = jnp.concatenate([edge_index[1], N + ar % (NPAD - N)])
    zeros1 = jnp.zeros((NPAD,), jnp.float32)
    zeros2 = jnp.zeros((NPAD, D), jnp.float32)

    deg_parts = _sc_degree(rows_h, zeros1).reshape(2, NPAD, 1)

    u, dinv = pl.pallas_call(
        _tc_scale_body,
        grid=(GRID,),
        in_specs=[
            pl.BlockSpec((2, BLK, 1), lambda i: (0, i, 0)),
            pl.BlockSpec((BLK, D), lambda i: (i, 0)),
        ],
        out_specs=[
            pl.BlockSpec((BLK, D), lambda i: (i, 0)),
            pl.BlockSpec((BLK, 1), lambda i: (i, 0)),
        ],
        out_shape=[
            jax.ShapeDtypeStruct((N, D), jnp.float32),
            jax.ShapeDtypeStruct((N, 1), jnp.float32),
        ],
    )(deg_parts, x)

    u_pad = jnp.concatenate([u, jnp.zeros((NPAD - N, D), jnp.float32)])
    s_parts = _sc_aggregate(u_pad, cols_a, rows_a, zeros2)

    out = pl.pallas_call(
        _tc_final_body,
        grid=(GRID,),
        in_specs=[
            pl.BlockSpec((2, BLK, D), lambda i: (0, i, 0)),
            pl.BlockSpec((BLK, D), lambda i: (i, 0)),
            pl.BlockSpec((BLK, 1), lambda i: (i, 0)),
            pl.BlockSpec((D, D), lambda i: (0, 0)),
            pl.BlockSpec((1, D), lambda i: (0, 0)),
        ],
        out_specs=pl.BlockSpec((BLK, D), lambda i: (i, 0)),
        out_shape=jax.ShapeDtypeStruct((N, D), jnp.float32),
    )(s_parts, u, dinv, W, b.reshape(1, D))

    return out
